# baseline (speedup 1.0000x reference)
import functools

import jax
import jax.numpy as jnp
from jax import lax
from jax.experimental import pallas as pl
from jax.experimental.pallas import tpu as pltpu
from jax.experimental.pallas import tpu_sc as plsc

_NW = 32
_CHUNK = 64


def _sc_gather(table, idx, off, n):
    d = table.shape[1]
    h = d // 2
    b_per_w = n // _NW
    n_chunks = b_per_w // _CHUNK
    mesh = plsc.VectorSubcoreMesh(core_axis_name="c", subcore_axis_name="s")

    @functools.partial(
        pl.kernel,
        mesh=mesh,
        out_type=jax.ShapeDtypeStruct((n, h), jnp.int32),
        scratch_types=[
            pltpu.VMEM((b_per_w,), jnp.int32),
            pltpu.VMEM((2, _CHUNK, d), jnp.float32),
            pltpu.VMEM((_CHUNK, h), jnp.int32),
            pltpu.SemaphoreType.DMA,
        ],
    )
    def k(table_hbm, idx_hbm, out_hbm, idx_v, rows_v, pack_v, gsem):
        wid = lax.axis_index("s") * 2 + lax.axis_index("c")
        base = wid * b_per_w
        pltpu.sync_copy(idx_hbm.at[pl.ds(off + base, b_per_w)], idx_v)
        pltpu.async_copy(
            table_hbm.at[idx_v.at[pl.ds(0, _CHUNK)]], rows_v.at[0], gsem)
        himask = jnp.int32(-65536)

        def body(kk, _):
            slot = lax.rem(kk, 2)
            nxt = lax.rem(kk + 1, 2)
            pltpu.make_async_copy(
                table_hbm.at[idx_v.at[pl.ds(0, _CHUNK)]],
                rows_v.at[slot], gsem).wait()

            @pl.when(kk + 1 < n_chunks)
            def _start_next():
                pltpu.async_copy(
                    table_hbm.at[idx_v.at[pl.ds((kk + 1) * _CHUNK, _CHUNK)]],
                    rows_v.at[nxt], gsem)

            def pack_chunk(src):
                @plsc.parallel_loop(0, _CHUNK, unroll=2)
                def _(r):
                    for g in range(h // 16):
                        c = g * 16
                        a = lax.bitcast_convert_type(
                            src[r, pl.ds(c, 16)], jnp.int32)
                        bb = lax.bitcast_convert_type(
                            src[r, pl.ds(h + c, 16)], jnp.int32)
                        lo = lax.shift_right_logical(a + 0x8000, 16)
                        hi = (bb + 0x8000) & himask
                        pack_v[r, pl.ds(c, 16)] = lo | hi

            @pl.when(slot == 0)
            def _pack0():
                pack_chunk(rows_v.at[0])

            @pl.when(slot != 0)
            def _pack1():
                pack_chunk(rows_v.at[1])
            pltpu.sync_copy(
                pack_v, out_hbm.at[pl.ds(base + kk * _CHUNK, _CHUNK)])
            return 0

        lax.fori_loop(0, n_chunks, body, 0)

    return k(table, idx)


def _tc_addln_slice(prev, words, pos_emb, type_pad, tids3, gamma2, beta2,
                    n_total, blk0):
    n, h = words.shape
    seq, d = pos_emb.shape
    t_blk = 512
    grid = n // t_blk
    blocks_per_seq = max(seq // t_blk, 1)

    def body(*refs):
        if prev is None:
            w_ref, p_ref, t_ref, id_ref, g_ref, b_ref, o_ref = refs
        else:
            _, w_ref, p_ref, t_ref, id_ref, g_ref, b_ref, o_ref = refs
        i = pl.program_id(0)
        p = p_ref[pl.ds(lax.rem(i, blocks_per_seq) * min(t_blk, seq), t_blk), :]
        w = w_ref[...]
        w_lo = lax.bitcast_convert_type(
            lax.shift_left(w, 16), jnp.float32)
        w_hi = lax.bitcast_convert_type(
            w & jnp.int32(-65536), jnp.float32)
        ids2 = id_ref[0].astype(jnp.bfloat16)
        r_io = lax.broadcasted_iota(jnp.int32, (t_blk, t_blk), 0)
        c_io = lax.broadcasted_iota(jnp.int32, (t_blk, t_blk), 1)
        a = (r_io == c_io).astype(jnp.bfloat16) * ids2
        delta = jnp.broadcast_to(
            (t_ref[1:2, :] - t_ref[0:1, :]).astype(jnp.bfloat16), (t_blk, d))
        tv = jnp.dot(a, delta, preferred_element_type=jnp.float32)
        v_lo = w_lo + p[:, :h] + t_ref[0:1, :h] + tv[:, :h]
        v_hi = w_hi + p[:, h:] + t_ref[0:1, h:] + tv[:, h:]
        mean = (jnp.sum(v_lo, axis=1, keepdims=True)
                + jnp.sum(v_hi, axis=1, keepdims=True)) * (1.0 / d)
        c_lo = v_lo - mean
        c_hi = v_hi - mean
        var = (jnp.sum(c_lo * c_lo, axis=1, keepdims=True)
               + jnp.sum(c_hi * c_hi, axis=1, keepdims=True)) * (1.0 / d)
        s = lax.rsqrt(var + 1e-12)
        o_ref[:, :h] = c_lo * s * g_ref[:, :h] + b_ref[:, :h]
        o_ref[:, h:] = c_hi * s * g_ref[:, h:] + b_ref[:, h:]

    in_specs = [
        pl.BlockSpec((t_blk, h), lambda i: (i, 0)),
        pl.BlockSpec((seq, d), lambda i: (0, 0)),
        pl.BlockSpec((8, d), lambda i: (0, 0)),
        pl.BlockSpec((1, 1, t_blk), lambda i: (blk0 + i, 0, 0)),
        pl.BlockSpec((1, d), lambda i: (0, 0)),
        pl.BlockSpec((1, d), lambda i: (0, 0)),
    ]
    args = (words, pos_emb, type_pad, tids3, gamma2, beta2)
    aliases = {}
    if prev is not None:
        in_specs = [pl.BlockSpec(memory_space=pl.ANY)] + in_specs
        args = (prev,) + args
        aliases = {0: 0}
    return pl.pallas_call(
        body,
        grid=(grid,),
        in_specs=in_specs,
        out_specs=pl.BlockSpec((t_blk, d), lambda i: (blk0 + i, 0)),
        out_shape=jax.ShapeDtypeStruct((n_total, d), jnp.float32),
        input_output_aliases=aliases,
    )(*args)


def kernel(input_ids, token_type_ids, attention_mask, word_embeddings,
           position_embeddings, token_type_embeddings, ln_gamma, ln_beta):
    b, l = input_ids.shape
    d = word_embeddings.shape[1]
    n = b * l
    slice_sizes = (2048, 10240, 10240, 10240)
    t_blk = 512
    ids_flat = input_ids.reshape(-1).astype(jnp.int32)
    tids3 = token_type_ids.reshape(n // t_blk, 1, t_blk).astype(jnp.int32)
    type_pad = jnp.zeros((8, d), jnp.float32).at[:2].set(token_type_embeddings)
    gamma2 = ln_gamma.reshape(1, d)
    beta2 = ln_beta.reshape(1, d)

    offs = [sum(slice_sizes[:i]) for i in range(len(slice_sizes))]
    word_slices = [
        _sc_gather(word_embeddings, ids_flat, off, sz)
        for off, sz in zip(offs, slice_sizes)
    ]
    out = None
    for ws, off in zip(word_slices, offs):
        out = _tc_addln_slice(out, ws, position_embeddings,
                              type_pad, tids3, gamma2, beta2,
                              n_total=n, blk0=off // t_blk)
    return (out.reshape(b, l, d), attention_mask)


# speedup vs baseline: 1.0138x; 1.0138x over previous
import functools

import jax
import jax.numpy as jnp
from jax import lax
from jax.experimental import pallas as pl
from jax.experimental.pallas import tpu as pltpu
from jax.experimental.pallas import tpu_sc as plsc

_NW = 32
_CHUNK = 64


def _sc_gather(table, idx, off, n):
    d = table.shape[1]
    h = d // 2
    b_per_w = n // _NW
    n_chunks = b_per_w // _CHUNK
    mesh = plsc.VectorSubcoreMesh(core_axis_name="c", subcore_axis_name="s")

    @functools.partial(
        pl.kernel,
        mesh=mesh,
        out_type=jax.ShapeDtypeStruct((n, h), jnp.int32),
        scratch_types=[
            pltpu.VMEM((b_per_w,), jnp.int32),
            pltpu.VMEM((2, _CHUNK, d), jnp.float32),
            pltpu.VMEM((_CHUNK, h), jnp.int32),
            pltpu.SemaphoreType.DMA,
        ],
    )
    def k(table_hbm, idx_hbm, out_hbm, idx_v, rows_v, pack_v, gsem):
        wid = lax.axis_index("s") * 2 + lax.axis_index("c")
        base = wid * b_per_w
        pltpu.sync_copy(idx_hbm.at[pl.ds(off + base, b_per_w)], idx_v)
        pltpu.async_copy(
            table_hbm.at[idx_v.at[pl.ds(0, _CHUNK)]], rows_v.at[0], gsem)
        himask = jnp.int32(-65536)

        def body(kk, _):
            slot = lax.rem(kk, 2)
            nxt = lax.rem(kk + 1, 2)
            pltpu.make_async_copy(
                table_hbm.at[idx_v.at[pl.ds(0, _CHUNK)]],
                rows_v.at[slot], gsem).wait()

            @pl.when(kk + 1 < n_chunks)
            def _start_next():
                pltpu.async_copy(
                    table_hbm.at[idx_v.at[pl.ds((kk + 1) * _CHUNK, _CHUNK)]],
                    rows_v.at[nxt], gsem)

            def pack_chunk(src):
                @plsc.parallel_loop(0, _CHUNK, unroll=2)
                def _(r):
                    for g in range(h // 16):
                        c = g * 16
                        a = lax.bitcast_convert_type(
                            src[r, pl.ds(c, 16)], jnp.int32)
                        bb = lax.bitcast_convert_type(
                            src[r, pl.ds(h + c, 16)], jnp.int32)
                        lo = lax.shift_right_logical(a + 0x8000, 16)
                        hi = (bb + 0x8000) & himask
                        pack_v[r, pl.ds(c, 16)] = lo | hi

            @pl.when(slot == 0)
            def _pack0():
                pack_chunk(rows_v.at[0])

            @pl.when(slot != 0)
            def _pack1():
                pack_chunk(rows_v.at[1])
            pltpu.sync_copy(
                pack_v, out_hbm.at[pl.ds(base + kk * _CHUNK, _CHUNK)])
            return 0

        lax.fori_loop(0, n_chunks, body, 0)

    return k(table, idx)


def _tc_addln_slice(prev, words, pos_emb, type_pad, tids3, gamma2, beta2,
                    n_total, blk0):
    n, h = words.shape
    seq, d = pos_emb.shape
    t_blk = 512
    grid = n // t_blk
    blocks_per_seq = max(seq // t_blk, 1)

    def body(*refs):
        if prev is None:
            w_ref, p_ref, t_ref, id_ref, g_ref, b_ref, o_ref = refs
        else:
            _, w_ref, p_ref, t_ref, id_ref, g_ref, b_ref, o_ref = refs
        i = pl.program_id(0)
        p = p_ref[pl.ds(lax.rem(i, blocks_per_seq) * min(t_blk, seq), t_blk), :]
        w = w_ref[...]
        w_lo = lax.bitcast_convert_type(
            lax.shift_left(w, 16), jnp.float32)
        w_hi = lax.bitcast_convert_type(
            w & jnp.int32(-65536), jnp.float32)
        ids2 = id_ref[0].astype(jnp.bfloat16)
        r_io = lax.broadcasted_iota(jnp.int32, (t_blk, t_blk), 0)
        c_io = lax.broadcasted_iota(jnp.int32, (t_blk, t_blk), 1)
        a = (r_io == c_io).astype(jnp.bfloat16) * ids2
        delta = jnp.broadcast_to(
            (t_ref[1:2, :] - t_ref[0:1, :]).astype(jnp.bfloat16), (t_blk, d))
        tv = jnp.dot(a, delta, preferred_element_type=jnp.float32)
        v_lo = w_lo + p[:, :h] + t_ref[0:1, :h] + tv[:, :h]
        v_hi = w_hi + p[:, h:] + t_ref[0:1, h:] + tv[:, h:]
        mean = (jnp.sum(v_lo, axis=1, keepdims=True)
                + jnp.sum(v_hi, axis=1, keepdims=True)) * (1.0 / d)
        c_lo = v_lo - mean
        c_hi = v_hi - mean
        var = (jnp.sum(c_lo * c_lo, axis=1, keepdims=True)
               + jnp.sum(c_hi * c_hi, axis=1, keepdims=True)) * (1.0 / d)
        s = lax.rsqrt(var + 1e-12)
        o_ref[:, :h] = c_lo * s * g_ref[:, :h] + b_ref[:, :h]
        o_ref[:, h:] = c_hi * s * g_ref[:, h:] + b_ref[:, h:]

    in_specs = [
        pl.BlockSpec((t_blk, h), lambda i: (i, 0)),
        pl.BlockSpec((seq, d), lambda i: (0, 0)),
        pl.BlockSpec((8, d), lambda i: (0, 0)),
        pl.BlockSpec((1, 1, t_blk), lambda i: (blk0 + i, 0, 0)),
        pl.BlockSpec((1, d), lambda i: (0, 0)),
        pl.BlockSpec((1, d), lambda i: (0, 0)),
    ]
    args = (words, pos_emb, type_pad, tids3, gamma2, beta2)
    aliases = {}
    if prev is not None:
        in_specs = [pl.BlockSpec(memory_space=pl.ANY)] + in_specs
        args = (prev,) + args
        aliases = {0: 0}
    return pl.pallas_call(
        body,
        grid=(grid,),
        in_specs=in_specs,
        out_specs=pl.BlockSpec((t_blk, d), lambda i: (blk0 + i, 0)),
        out_shape=jax.ShapeDtypeStruct((n_total, d), jnp.float32),
        input_output_aliases=aliases,
    )(*args)


def kernel(input_ids, token_type_ids, attention_mask, word_embeddings,
           position_embeddings, token_type_embeddings, ln_gamma, ln_beta):
    b, l = input_ids.shape
    d = word_embeddings.shape[1]
    n = b * l
    slice_sizes = (8192, 8192, 8192, 8192)
    t_blk = 512
    ids_flat = input_ids.reshape(-1).astype(jnp.int32)
    tids3 = token_type_ids.reshape(n // t_blk, 1, t_blk).astype(jnp.int32)
    type_pad = jnp.zeros((8, d), jnp.float32).at[:2].set(token_type_embeddings)
    gamma2 = ln_gamma.reshape(1, d)
    beta2 = ln_beta.reshape(1, d)

    offs = [sum(slice_sizes[:i]) for i in range(len(slice_sizes))]
    word_slices = [
        _sc_gather(word_embeddings, ids_flat, off, sz)
        for off, sz in zip(offs, slice_sizes)
    ]
    out = None
    for ws, off in zip(word_slices, offs):
        out = _tc_addln_slice(out, ws, position_embeddings,
                              type_pad, tids3, gamma2, beta2,
                              n_total=n, blk0=off // t_blk)
    return (out.reshape(b, l, d), attention_mask)
